# Initial kernel scaffold; baseline (speedup 1.0000x reference)
import jax
import jax.numpy as jnp
from jax.experimental import pallas as pl


def kernel(x, edge_index, edge_attr, W1_rel, b1_rel, W1_root, W2_rel, b2_rel, W2_root):
    raise NotImplementedError("write your pallas kernel here")


# speedup vs baseline: 1.4474x; 1.4474x over previous
import functools

import jax
import jax.numpy as jnp
from jax import lax
from jax.experimental import pallas as pl
from jax.experimental.pallas import tpu as pltpu
from jax.experimental.pallas import tpu_sc as plsc

N = 10000
D = 256
E = 160000
NW = 32
R = 313
NPAD = NW * R
CHUNK = 3200
NCHUNK = E // CHUNK
FLUSH = 3200
CAP = E + FLUSH
G = 128
ACC = (R + 1) * D

_mesh = plsc.VectorSubcoreMesh(core_axis_name="c", subcore_axis_name="s")
_sc_params = pltpu.CompilerParams(needs_layout_passes=False)


def _wid():
    return lax.axis_index("s") * 2 + lax.axis_index("c")


@functools.partial(
    pl.kernel,
    out_type=(
        jax.ShapeDtypeStruct((NW * CAP,), jnp.int32),
        jax.ShapeDtypeStruct((NW * 16,), jnp.int32),
    ),
    mesh=_mesh,
    compiler_params=_sc_params,
    scratch_types=[
        pltpu.VMEM((CHUNK,), jnp.int32),
        pltpu.VMEM((CHUNK,), jnp.int32),
        pltpu.VMEM((FLUSH + CHUNK,), jnp.int32),
        pltpu.VMEM((16,), jnp.int32),
        pltpu.SemaphoreType.DMA,
    ],
)
def _sc_bin(src_hbm, dst_hbm, packed_hbm, counts_hbm, sbuf, dbuf, buf, cntb, sem):
    wid = _wid()
    lo = wid * R
    iota = lax.iota(jnp.int32, 16)
    sent = jnp.left_shift(jnp.zeros((16,), jnp.int32) + R, 14) | (iota * 619 + wid * 3)

    def init(i, _):
        buf[pl.ds(i * 16, 16)] = sent
        return 0

    lax.fori_loop(0, (FLUSH + CHUNK) // 16, init, 0)

    def chunk_body(c, carry):
        off_v, flushes = carry
        pltpu.sync_copy(src_hbm.at[pl.ds(c * CHUNK, CHUNK)], sbuf)
        pltpu.sync_copy(dst_hbm.at[pl.ds(c * CHUNK, CHUNK)], dbuf)

        def scan_body(i, off_v):
            d = dbuf[pl.ds(i * 16, 16)]
            s = sbuf[pl.ds(i * 16, 16)]
            dl = d - lo
            m = (dl >= 0) & (dl < R)
            pk = jnp.left_shift(dl, 14) | s
            pos = plsc.cumsum(jnp.where(m, 1, 0).astype(jnp.int32))
            plsc.store_scatter(buf, [off_v + pos - 1], pk, mask=m)
            return off_v + plsc.all_reduce_population_count(m)

        off_v = lax.fori_loop(0, CHUNK // 16, scan_body, off_v)
        off_s = jnp.max(off_v)
        do_flush = off_s >= FLUSH

        @pl.when(do_flush)
        def _():
            pltpu.sync_copy(
                buf.at[pl.ds(0, FLUSH)],
                packed_hbm.at[pl.ds(wid * CAP + flushes * FLUSH, FLUSH)],
            )
            for k in range(CHUNK // 16):
                buf[pl.ds(k * 16, 16)] = buf[pl.ds(FLUSH + k * 16, 16)]

        off_v = jnp.where(do_flush, off_v - FLUSH, off_v)
        flushes = jnp.where(do_flush, flushes + 1, flushes)
        return off_v, flushes

    off_v, flushes = lax.fori_loop(
        0, NCHUNK, chunk_body, (jnp.zeros((16,), jnp.int32), jnp.int32(0))
    )
    plsc.store_scatter(buf, [off_v + iota], sent)
    pltpu.sync_copy(
        buf.at[pl.ds(0, FLUSH)],
        packed_hbm.at[pl.ds(wid * CAP + flushes * FLUSH, FLUSH)],
    )
    cntb[...] = jnp.zeros((16,), jnp.int32) + (flushes + 1) * FLUSH
    pltpu.sync_copy(cntb, counts_hbm.at[pl.ds(wid * 16, 16)])


@functools.partial(
    pl.kernel,
    out_type=jax.ShapeDtypeStruct((NPAD * D,), jnp.float32),
    mesh=_mesh,
    compiler_params=_sc_params,
    scratch_types=[
        pltpu.VMEM((ACC,), jnp.float32),
        pltpu.VMEM((G, D), jnp.float32),
        pltpu.VMEM((G,), jnp.int32),
        pltpu.VMEM((G,), jnp.int32),
        pltpu.VMEM((G,), jnp.int32),
        pltpu.VMEM((16,), jnp.int32),
        pltpu.SemaphoreType.DMA,
    ],
)
def _sc_segmax(x_hbm, packed_hbm, counts_hbm, out_hbm, acc, rows, pk, srcs, dls, cntb, sem):
    wid = _wid()
    iota = lax.iota(jnp.int32, 16)
    neginf = jnp.full((16,), -jnp.inf, jnp.float32)

    def init(i, _):
        acc[pl.ds(i * 16, 16)] = neginf
        return 0

    lax.fori_loop(0, ACC // 16, init, 0)

    pltpu.sync_copy(counts_hbm.at[pl.ds(wid * 16, 16)], cntb)
    n = jnp.max(cntb[...])

    def batch(b, _):
        pltpu.sync_copy(packed_hbm.at[pl.ds(wid * CAP + b * G, G)], pk)
        for i in range(G // 16):
            v = pk[pl.ds(i * 16, 16)]
            srcs[pl.ds(i * 16, 16)] = v & 16383
            dls[pl.ds(i * 16, 16)] = lax.shift_right_logical(v, 14)
        pltpu.async_copy(x_hbm.at[srcs], rows, sem).wait()

        def edge(e, _):
            e_spl = jnp.zeros((16,), jnp.int32) + e
            d_spl = plsc.load_gather(dls, [e_spl])
            base = d_spl * D + iota
            for j in range(D // 16):
                r = plsc.load_gather(rows, [e_spl, iota + (j * 16)])
                addr = base + (j * 16)
                a = plsc.load_gather(acc, [addr])
                plsc.store_scatter(acc, [addr], jnp.maximum(a, r))
            return 0

        lax.fori_loop(0, G, edge, 0)
        return 0

    lax.fori_loop(0, n // G, batch, 0)
    pltpu.sync_copy(acc.at[pl.ds(0, R * D)], out_hbm.at[pl.ds(wid * R * D, R * D)])


_BN = 2504


def _dense_body(agg_ref, x_ref, wrel_ref, wroot_ref, b_ref, o_ref):
    a = agg_ref[...]
    a = jnp.where(jnp.isfinite(a), a, 0.0)
    h = jnp.dot(a, wrel_ref[...], preferred_element_type=jnp.float32)
    h = h + jnp.dot(x_ref[...], wroot_ref[...], preferred_element_type=jnp.float32)
    h = h + b_ref[...]
    o_ref[...] = jnp.maximum(h, 0.0)


_dense = pl.pallas_call(
    _dense_body,
    grid=(NPAD // _BN,),
    in_specs=[
        pl.BlockSpec((_BN, D), lambda i: (i, 0)),
        pl.BlockSpec((_BN, D), lambda i: (i, 0)),
        pl.BlockSpec((D, D), lambda i: (0, 0)),
        pl.BlockSpec((D, D), lambda i: (0, 0)),
        pl.BlockSpec((1, D), lambda i: (0, 0)),
    ],
    out_specs=pl.BlockSpec((_BN, D), lambda i: (i, 0)),
    out_shape=jax.ShapeDtypeStruct((NPAD, D), jnp.float32),
)


def kernel(x, edge_index, edge_attr, W1_rel, b1_rel, W1_root, W2_rel, b2_rel, W2_root):
    del edge_attr
    src = edge_index[0]
    dst = edge_index[1]
    packed, counts = _sc_bin(src, dst)
    x_pad = jnp.pad(x, ((0, NPAD - N), (0, 0)))
    agg1 = _sc_segmax(x_pad, packed, counts).reshape(NPAD, D)
    h1 = _dense(agg1, x_pad, W1_rel.T, W1_root.T, b1_rel.reshape(1, D))
    agg2 = _sc_segmax(h1, packed, counts).reshape(NPAD, D)
    h2 = _dense(agg2, h1, W2_rel.T, W2_root.T, b2_rel.reshape(1, D))
    return h2[:N]


# speedup vs baseline: 3.7046x; 2.5595x over previous
import functools

import jax
import jax.numpy as jnp
from jax import lax
from jax.experimental import pallas as pl
from jax.experimental.pallas import tpu as pltpu
from jax.experimental.pallas import tpu_sc as plsc

N = 10000
D = 256
E = 160000
NW = 32
R = 313
NPAD = NW * R
CHUNK = 3200
NCHUNK = E // CHUNK
FLUSH = 640
RING = CHUNK + FLUSH
CAP = E + FLUSH
SUP = 640
G = 64
NB = SUP // G
ACC = (R + 1) * D

_mesh = plsc.VectorSubcoreMesh(core_axis_name="c", subcore_axis_name="s")
_sc_params = pltpu.CompilerParams(needs_layout_passes=False)


def _wid():
    return lax.axis_index("s") * 2 + lax.axis_index("c")


def _splat(x):
    return jnp.zeros((16,), jnp.int32) + x


@functools.partial(
    pl.kernel,
    out_type=(
        jax.ShapeDtypeStruct((NW * CAP,), jnp.int32),
        jax.ShapeDtypeStruct((NW * 16,), jnp.int32),
    ),
    mesh=_mesh,
    compiler_params=_sc_params,
    scratch_types=[
        pltpu.VMEM((CHUNK,), jnp.int32),
        pltpu.VMEM((CHUNK,), jnp.int32),
        pltpu.VMEM((CHUNK,), jnp.int32),
        pltpu.VMEM((CHUNK,), jnp.int32),
        pltpu.VMEM((RING,), jnp.int32),
        pltpu.VMEM((16,), jnp.int32),
        pltpu.SemaphoreType.DMA,
        pltpu.SemaphoreType.DMA,
        pltpu.SemaphoreType.DMA,
        pltpu.SemaphoreType.DMA,
    ],
)
def _sc_bin(src_hbm, dst_hbm, packed_hbm, counts_hbm,
            sb0, sb1, db0, db1, buf, cntb, ss0, ss1, sd0, sd1):
    wid = _wid()
    lo = wid * R
    iota = lax.iota(jnp.int32, 16)
    sent = jnp.left_shift(_splat(R), 14) | (iota * 619 + wid * 3)
    sbufs, dbufs = (sb0, sb1), (db0, db1)
    ssems, dsems = (ss0, ss1), (sd0, sd1)

    pltpu.async_copy(src_hbm.at[pl.ds(0, CHUNK)], sb0, ss0)
    pltpu.async_copy(dst_hbm.at[pl.ds(0, CHUNK)], db0, sd0)

    def do_chunk(c, b, carry):
        off_v, infl_v, fpos, flushes = carry
        sb, db = sbufs[b], dbufs[b]
        pltpu.make_async_copy(src_hbm.at[pl.ds(c * CHUNK, CHUNK)], sb, ssems[b]).wait()
        pltpu.make_async_copy(dst_hbm.at[pl.ds(c * CHUNK, CHUNK)], db, dsems[b]).wait()

        @pl.when(c < NCHUNK - 1)
        def _():
            nxt = (c + 1) * CHUNK
            pltpu.async_copy(src_hbm.at[pl.ds(nxt, CHUNK)], sbufs[1 - b], ssems[1 - b])
            pltpu.async_copy(dst_hbm.at[pl.ds(nxt, CHUNK)], dbufs[1 - b], dsems[1 - b])

        def scan_body(i, carry2):
            off_v, infl_v = carry2
            d = db[pl.ds(i * 16, 16)]
            s = sb[pl.ds(i * 16, 16)]
            dl = d - lo
            m = (dl >= 0) & (dl < R)
            pk = jnp.left_shift(dl, 14) | s
            pos = plsc.cumsum(jnp.where(m, 1, 0).astype(jnp.int32))
            a = off_v + pos - 1
            a = jnp.where(a >= RING, a - RING, a)
            plsc.store_scatter(buf, [a], pk, mask=m)
            pc = plsc.all_reduce_population_count(m)
            off_v = off_v + pc
            off_v = jnp.where(off_v >= RING, off_v - RING, off_v)
            return off_v, infl_v + pc

        off_v, infl_v = lax.fori_loop(0, CHUNK // 16, scan_body, (off_v, infl_v))

        infl0 = jnp.max(infl_v)

        def fl_cond(st):
            return st[0] >= FLUSH

        def fl_body(st):
            infl, fpos, flushes = st
            pltpu.sync_copy(
                buf.at[pl.ds(pl.multiple_of(fpos, 8), FLUSH)],
                packed_hbm.at[pl.ds(pl.multiple_of(wid * CAP + flushes * FLUSH, 8), FLUSH)],
            )
            fpos = fpos + FLUSH
            fpos = jnp.where(fpos >= RING, fpos - RING, fpos)
            return infl - FLUSH, fpos, flushes + 1

        infl1, fpos, flushes = lax.while_loop(fl_cond, fl_body, (infl0, fpos, flushes))
        infl_v = infl_v - (infl0 - infl1)
        return off_v, infl_v, fpos, flushes

    def pair_body(p, carry):
        carry = do_chunk(p * 2, 0, carry)
        carry = do_chunk(p * 2 + 1, 1, carry)
        return carry

    off_v, infl_v, fpos, flushes = lax.fori_loop(
        0, NCHUNK // 2, pair_body,
        (jnp.zeros((16,), jnp.int32), jnp.zeros((16,), jnp.int32),
         jnp.int32(0), jnp.int32(0)),
    )

    infl_s = jnp.max(infl_v)
    target = ((infl_s + FLUSH - 1) // FLUSH) * FLUSH

    def seal_cond(st):
        return st[0] < target

    def seal_body(st):
        filled, off_v = st
        a = off_v + iota
        a = jnp.where(a >= RING, a - RING, a)
        plsc.store_scatter(buf, [a], sent)
        off_v = off_v + 16
        off_v = jnp.where(off_v >= RING, off_v - RING, off_v)
        return filled + 16, off_v

    lax.while_loop(seal_cond, seal_body, (infl_s, off_v))

    def tail_cond(st):
        return st[0] > 0

    def tail_body(st):
        left, fpos, flushes = st
        pltpu.sync_copy(
            buf.at[pl.ds(pl.multiple_of(fpos, 8), FLUSH)],
            packed_hbm.at[pl.ds(pl.multiple_of(wid * CAP + flushes * FLUSH, 8), FLUSH)],
        )
        fpos = fpos + FLUSH
        fpos = jnp.where(fpos >= RING, fpos - RING, fpos)
        return left - FLUSH, fpos, flushes + 1

    _, _, flushes = lax.while_loop(tail_cond, tail_body, (target, fpos, flushes))

    cntb[...] = _splat(flushes * FLUSH)
    pltpu.sync_copy(cntb, counts_hbm.at[pl.ds(wid * 16, 16)])


@functools.partial(
    pl.kernel,
    out_type=jax.ShapeDtypeStruct((NPAD * D,), jnp.float32),
    mesh=_mesh,
    compiler_params=_sc_params,
    scratch_types=[
        pltpu.VMEM((ACC,), jnp.float32),
        pltpu.VMEM((G, D), jnp.float32),
        pltpu.VMEM((G, D), jnp.float32),
        pltpu.VMEM((SUP,), jnp.int32),
        pltpu.VMEM((SUP,), jnp.int32),
        pltpu.VMEM((SUP,), jnp.int32),
        pltpu.VMEM((16,), jnp.int32),
        pltpu.SemaphoreType.DMA,
        pltpu.SemaphoreType.DMA,
    ],
)
def _sc_segmax(x_hbm, packed_hbm, counts_hbm, out_hbm,
               acc, rows0, rows1, pk, srcs, dls, cntb, sem0, sem1):
    wid = _wid()
    iota = lax.iota(jnp.int32, 16)
    neginf = jnp.full((16,), -jnp.inf, jnp.float32)
    rowbufs = (rows0, rows1)
    sems = (sem0, sem1)

    def init(i, _):
        acc[pl.ds(i * 16, 16)] = neginf
        return 0

    lax.fori_loop(0, ACC // 16, init, 0)

    pltpu.sync_copy(counts_hbm.at[pl.ds(wid * 16, 16)], cntb)
    n = jnp.max(cntb[...])

    def super_chunk(s, _):
        pltpu.sync_copy(packed_hbm.at[pl.ds(wid * CAP + s * SUP, SUP)], pk)
        for i in range(SUP // 16):
            v = pk[pl.ds(i * 16, 16)]
            srcs[pl.ds(i * 16, 16)] = v & 16383
            dls[pl.ds(i * 16, 16)] = lax.shift_right_logical(v, 14)
        pltpu.async_copy(x_hbm.at[srcs.at[pl.ds(0, G)]], rows0, sem0)

        def do_batch(t, b):
            rows = rowbufs[b]
            pltpu.make_async_copy(
                x_hbm.at[srcs.at[pl.ds(t * G, G)]], rows, sems[b]
            ).wait()

            @pl.when(t < NB - 1)
            def _():
                pltpu.async_copy(
                    x_hbm.at[srcs.at[pl.ds((t + 1) * G, G)]],
                    rowbufs[1 - b], sems[1 - b],
                )

            def edge(e, _):
                d_spl = plsc.load_gather(dls, [_splat(t * G + e)])
                base = d_spl * D + iota
                for half in range(2):
                    js = [half * 8 + j for j in range(8)]
                    rs = [rows[e, pl.ds(j * 16, 16)] for j in js]
                    ads = [base + j * 16 for j in js]
                    avs = [plsc.load_gather(acc, [ad]) for ad in ads]
                    for k in range(8):
                        plsc.store_scatter(acc, [ads[k]], jnp.maximum(avs[k], rs[k]))
                return 0

            lax.fori_loop(0, G, edge, 0)

        def batch_pair(tp, _):
            do_batch(tp * 2, 0)
            do_batch(tp * 2 + 1, 1)
            return 0

        lax.fori_loop(0, NB // 2, batch_pair, 0)
        return 0

    lax.fori_loop(0, n // SUP, super_chunk, 0)
    pltpu.sync_copy(acc.at[pl.ds(0, R * D)], out_hbm.at[pl.ds(wid * R * D, R * D)])


_BN = 2504


def _dense_body(agg_ref, x_ref, wrel_ref, wroot_ref, b_ref, o_ref):
    a = agg_ref[...]
    a = jnp.where(jnp.isfinite(a), a, 0.0)
    h = jnp.dot(a, wrel_ref[...], preferred_element_type=jnp.float32)
    h = h + jnp.dot(x_ref[...], wroot_ref[...], preferred_element_type=jnp.float32)
    h = h + b_ref[...]
    o_ref[...] = jnp.maximum(h, 0.0)


_dense = pl.pallas_call(
    _dense_body,
    grid=(NPAD // _BN,),
    in_specs=[
        pl.BlockSpec((_BN, D), lambda i: (i, 0)),
        pl.BlockSpec((_BN, D), lambda i: (i, 0)),
        pl.BlockSpec((D, D), lambda i: (0, 0)),
        pl.BlockSpec((D, D), lambda i: (0, 0)),
        pl.BlockSpec((1, D), lambda i: (0, 0)),
    ],
    out_specs=pl.BlockSpec((_BN, D), lambda i: (i, 0)),
    out_shape=jax.ShapeDtypeStruct((NPAD, D), jnp.float32),
)


def kernel(x, edge_index, edge_attr, W1_rel, b1_rel, W1_root, W2_rel, b2_rel, W2_root):
    del edge_attr
    src = edge_index[0]
    dst = edge_index[1]
    packed, counts = _sc_bin(src, dst)
    x_pad = jnp.pad(x, ((0, NPAD - N), (0, 0)))
    agg1 = _sc_segmax(x_pad, packed, counts).reshape(NPAD, D)
    h1 = _dense(agg1, x_pad, W1_rel.T, W1_root.T, b1_rel.reshape(1, D))
    agg2 = _sc_segmax(h1, packed, counts).reshape(NPAD, D)
    h2 = _dense(agg2, h1, W2_rel.T, W2_root.T, b2_rel.reshape(1, D))
    return h2[:N]


# speedup vs baseline: 4.3694x; 1.1794x over previous
import functools

import jax
import jax.numpy as jnp
from jax import lax
from jax.experimental import pallas as pl
from jax.experimental.pallas import tpu as pltpu
from jax.experimental.pallas import tpu_sc as plsc

N = 10000
D = 256
E = 160000
NW = 32
R = 313
NPAD = NW * R
CHUNK = 3200
NCHUNK = E // CHUNK
FLUSH = 640
RING = CHUNK + FLUSH
CAP = E + FLUSH
SUP = 640
G = 64
NB = SUP // G
ACC = (R + 1) * D

_mesh = plsc.VectorSubcoreMesh(core_axis_name="c", subcore_axis_name="s")
_sc_params = pltpu.CompilerParams(needs_layout_passes=False)


def _wid():
    return lax.axis_index("s") * 2 + lax.axis_index("c")


def _splat(x):
    return jnp.zeros((16,), jnp.int32) + x


@functools.partial(
    pl.kernel,
    out_type=(
        jax.ShapeDtypeStruct((NW * CAP,), jnp.int32),
        jax.ShapeDtypeStruct((NW * 16,), jnp.int32),
    ),
    mesh=_mesh,
    compiler_params=_sc_params,
    scratch_types=[
        pltpu.VMEM((CHUNK,), jnp.int32),
        pltpu.VMEM((CHUNK,), jnp.int32),
        pltpu.VMEM((CHUNK,), jnp.int32),
        pltpu.VMEM((CHUNK,), jnp.int32),
        pltpu.VMEM((RING,), jnp.int32),
        pltpu.VMEM((16,), jnp.int32),
        pltpu.SemaphoreType.DMA,
        pltpu.SemaphoreType.DMA,
        pltpu.SemaphoreType.DMA,
        pltpu.SemaphoreType.DMA,
    ],
)
def _sc_bin(src_hbm, dst_hbm, packed_hbm, counts_hbm,
            sb0, sb1, db0, db1, buf, cntb, ss0, ss1, sd0, sd1):
    wid = _wid()
    lo = wid * R
    iota = lax.iota(jnp.int32, 16)
    sent = jnp.left_shift(_splat(R), 14) | (iota * 619 + wid * 3)
    sbufs, dbufs = (sb0, sb1), (db0, db1)
    ssems, dsems = (ss0, ss1), (sd0, sd1)

    pltpu.async_copy(src_hbm.at[pl.ds(0, CHUNK)], sb0, ss0)
    pltpu.async_copy(dst_hbm.at[pl.ds(0, CHUNK)], db0, sd0)

    def do_chunk(c, b, carry):
        off_v, infl_v, fpos, flushes = carry
        sb, db = sbufs[b], dbufs[b]
        pltpu.make_async_copy(src_hbm.at[pl.ds(c * CHUNK, CHUNK)], sb, ssems[b]).wait()
        pltpu.make_async_copy(dst_hbm.at[pl.ds(c * CHUNK, CHUNK)], db, dsems[b]).wait()

        @pl.when(c < NCHUNK - 1)
        def _():
            nxt = (c + 1) * CHUNK
            pltpu.async_copy(src_hbm.at[pl.ds(nxt, CHUNK)], sbufs[1 - b], ssems[1 - b])
            pltpu.async_copy(dst_hbm.at[pl.ds(nxt, CHUNK)], dbufs[1 - b], dsems[1 - b])

        def scan_body(i, carry2):
            off_v, infl_v = carry2
            ds_ = [db[pl.ds((i * 4 + u) * 16, 16)] for u in range(4)]
            ss_ = [sb[pl.ds((i * 4 + u) * 16, 16)] for u in range(4)]
            dls_ = [d - lo for d in ds_]
            ms_ = [(dl >= 0) & (dl < R) for dl in dls_]
            pks_ = [jnp.left_shift(dl, 14) | s for dl, s in zip(dls_, ss_)]
            poss = [plsc.cumsum(jnp.where(m, 1, 0).astype(jnp.int32)) for m in ms_]
            pcs = [plsc.all_reduce_population_count(m) for m in ms_]
            for u in range(4):
                a = off_v + poss[u] - 1
                a = jnp.where(a >= RING, a - RING, a)
                plsc.store_scatter(buf, [a], pks_[u], mask=ms_[u])
                off_v = off_v + pcs[u]
                off_v = jnp.where(off_v >= RING, off_v - RING, off_v)
                infl_v = infl_v + pcs[u]
            return off_v, infl_v

        off_v, infl_v = lax.fori_loop(0, CHUNK // 64, scan_body, (off_v, infl_v))

        infl0 = jnp.max(infl_v)

        def fl_cond(st):
            return st[0] >= FLUSH

        def fl_body(st):
            infl, fpos, flushes = st
            pltpu.sync_copy(
                buf.at[pl.ds(pl.multiple_of(fpos, 8), FLUSH)],
                packed_hbm.at[pl.ds(pl.multiple_of(wid * CAP + flushes * FLUSH, 8), FLUSH)],
            )
            fpos = fpos + FLUSH
            fpos = jnp.where(fpos >= RING, fpos - RING, fpos)
            return infl - FLUSH, fpos, flushes + 1

        infl1, fpos, flushes = lax.while_loop(fl_cond, fl_body, (infl0, fpos, flushes))
        infl_v = infl_v - (infl0 - infl1)
        return off_v, infl_v, fpos, flushes

    def pair_body(p, carry):
        carry = do_chunk(p * 2, 0, carry)
        carry = do_chunk(p * 2 + 1, 1, carry)
        return carry

    off_v, infl_v, fpos, flushes = lax.fori_loop(
        0, NCHUNK // 2, pair_body,
        (jnp.zeros((16,), jnp.int32), jnp.zeros((16,), jnp.int32),
         jnp.int32(0), jnp.int32(0)),
    )

    infl_s = jnp.max(infl_v)
    target = ((infl_s + FLUSH - 1) // FLUSH) * FLUSH

    def seal_cond(st):
        return st[0] < target

    def seal_body(st):
        filled, off_v = st
        a = off_v + iota
        a = jnp.where(a >= RING, a - RING, a)
        plsc.store_scatter(buf, [a], sent)
        off_v = off_v + 16
        off_v = jnp.where(off_v >= RING, off_v - RING, off_v)
        return filled + 16, off_v

    lax.while_loop(seal_cond, seal_body, (infl_s, off_v))

    def tail_cond(st):
        return st[0] > 0

    def tail_body(st):
        left, fpos, flushes = st
        pltpu.sync_copy(
            buf.at[pl.ds(pl.multiple_of(fpos, 8), FLUSH)],
            packed_hbm.at[pl.ds(pl.multiple_of(wid * CAP + flushes * FLUSH, 8), FLUSH)],
        )
        fpos = fpos + FLUSH
        fpos = jnp.where(fpos >= RING, fpos - RING, fpos)
        return left - FLUSH, fpos, flushes + 1

    _, _, flushes = lax.while_loop(tail_cond, tail_body, (target, fpos, flushes))

    cntb[...] = _splat(flushes * FLUSH)
    pltpu.sync_copy(cntb, counts_hbm.at[pl.ds(wid * 16, 16)])


@functools.partial(
    pl.kernel,
    out_type=jax.ShapeDtypeStruct((NPAD * D,), jnp.float32),
    mesh=_mesh,
    compiler_params=_sc_params,
    scratch_types=[
        pltpu.VMEM((ACC,), jnp.float32),
        pltpu.VMEM((G, D), jnp.float32),
        pltpu.VMEM((G, D), jnp.float32),
        pltpu.VMEM((SUP,), jnp.int32),
        pltpu.VMEM((SUP,), jnp.int32),
        pltpu.VMEM((SUP,), jnp.int32),
        pltpu.VMEM((16,), jnp.int32),
        pltpu.SemaphoreType.DMA,
        pltpu.SemaphoreType.DMA,
    ],
)
def _sc_segmax(x_hbm, packed_hbm, counts_hbm, out_hbm,
               acc, rows0, rows1, pk, srcs, dls, cntb, sem0, sem1):
    wid = _wid()
    iota = lax.iota(jnp.int32, 16)
    neginf = jnp.full((16,), -jnp.inf, jnp.float32)
    rowbufs = (rows0, rows1)
    sems = (sem0, sem1)

    def init(i, _):
        acc[pl.ds(i * 16, 16)] = neginf
        return 0

    lax.fori_loop(0, ACC // 16, init, 0)

    pltpu.sync_copy(counts_hbm.at[pl.ds(wid * 16, 16)], cntb)
    n = jnp.max(cntb[...])

    def super_chunk(s, _):
        pltpu.sync_copy(packed_hbm.at[pl.ds(wid * CAP + s * SUP, SUP)], pk)
        for i in range(SUP // 16):
            v = pk[pl.ds(i * 16, 16)]
            srcs[pl.ds(i * 16, 16)] = v & 16383
            dls[pl.ds(i * 16, 16)] = lax.shift_right_logical(v, 14)
        pltpu.async_copy(x_hbm.at[srcs.at[pl.ds(0, G)]], rows0, sem0)

        def do_batch(t, b):
            rows = rowbufs[b]
            pltpu.make_async_copy(
                x_hbm.at[srcs.at[pl.ds(t * G, G)]], rows, sems[b]
            ).wait()

            @pl.when(t < NB - 1)
            def _():
                pltpu.async_copy(
                    x_hbm.at[srcs.at[pl.ds((t + 1) * G, G)]],
                    rowbufs[1 - b], sems[1 - b],
                )

            def edge(e, _):
                d_spl = plsc.load_gather(dls, [_splat(t * G + e)])
                base = d_spl * D + iota
                for half in range(2):
                    js = [half * 8 + j for j in range(8)]
                    rs = [rows[e, pl.ds(j * 16, 16)] for j in js]
                    ads = [base + j * 16 for j in js]
                    avs = [plsc.load_gather(acc, [ad]) for ad in ads]
                    for k in range(8):
                        plsc.store_scatter(acc, [ads[k]], jnp.maximum(avs[k], rs[k]))
                return 0

            lax.fori_loop(0, G, edge, 0)

        def batch_pair(tp, _):
            do_batch(tp * 2, 0)
            do_batch(tp * 2 + 1, 1)
            return 0

        lax.fori_loop(0, NB // 2, batch_pair, 0)
        return 0

    lax.fori_loop(0, n // SUP, super_chunk, 0)
    pltpu.sync_copy(acc.at[pl.ds(0, R * D)], out_hbm.at[pl.ds(wid * R * D, R * D)])


_BN = 2504


def _dense_body(agg_ref, x_ref, wrel_ref, wroot_ref, b_ref, o_ref):
    a = agg_ref[...]
    a = jnp.where(jnp.isfinite(a), a, 0.0)
    h = jnp.dot(a, wrel_ref[...], preferred_element_type=jnp.float32)
    h = h + jnp.dot(x_ref[...], wroot_ref[...], preferred_element_type=jnp.float32)
    h = h + b_ref[...]
    o_ref[...] = jnp.maximum(h, 0.0)


_dense = pl.pallas_call(
    _dense_body,
    grid=(NPAD // _BN,),
    in_specs=[
        pl.BlockSpec((_BN, D), lambda i: (i, 0)),
        pl.BlockSpec((_BN, D), lambda i: (i, 0)),
        pl.BlockSpec((D, D), lambda i: (0, 0)),
        pl.BlockSpec((D, D), lambda i: (0, 0)),
        pl.BlockSpec((1, D), lambda i: (0, 0)),
    ],
    out_specs=pl.BlockSpec((_BN, D), lambda i: (i, 0)),
    out_shape=jax.ShapeDtypeStruct((NPAD, D), jnp.float32),
)


def kernel(x, edge_index, edge_attr, W1_rel, b1_rel, W1_root, W2_rel, b2_rel, W2_root):
    del edge_attr
    src = edge_index[0]
    dst = edge_index[1]
    packed, counts = _sc_bin(src, dst)
    x_pad = jnp.pad(x, ((0, NPAD - N), (0, 0)))
    agg1 = _sc_segmax(x_pad, packed, counts).reshape(NPAD, D)
    h1 = _dense(agg1, x_pad, W1_rel.T, W1_root.T, b1_rel.reshape(1, D))
    agg2 = _sc_segmax(h1, packed, counts).reshape(NPAD, D)
    h2 = _dense(agg2, h1, W2_rel.T, W2_root.T, b2_rel.reshape(1, D))
    return h2[:N]
